# baseline (speedup 1.0000x reference)
import functools
import math

import jax
import jax.numpy as jnp
import numpy as np
from jax import lax
from jax.experimental import pallas as pl
from jax.experimental.pallas import tpu as pltpu
from jax.experimental.pallas import tpu_sc as plsc

N = 10000
E = 320000
D = 128
NF = 128
NRBF = 50
NI = 3
CUT = 5.0
ZMAX = 400

T = 16384
DD = CUT / (T - 1)

NC = 2
NS = 16
NW = NC * NS
N_PAD = 10240
RPT = N_PAD // NS
C = 80
NROW = 4096
CPW = NROW // NW

EPW = E // NW
GCH = 2000
NG = EPW // GCH

NB = 2000
NGRID = N // NB

_mesh = plsc.VectorSubcoreMesh(core_axis_name="c", subcore_axis_name="s")


def _ssp(x):
    return jax.nn.softplus(x) - math.log(2.0)


def _geom_body(rx_h, ry_h, rz_h, ii_h, ij_h, lo_h,
               rx_v, ry_v, rz_v, ii_v, ij_v, lo_v):
    c = lax.axis_index("c")
    s = lax.axis_index("s")
    wid = s * NC + c
    pltpu.sync_copy(rx_h, rx_v)
    pltpu.sync_copy(ry_h, ry_v)
    pltpu.sync_copy(rz_h, rz_v)
    base0 = wid * EPW

    @pl.loop(0, NG)
    def _chunk(ch):
        base = base0 + ch * GCH
        pltpu.sync_copy(ii_h.at[pl.ds(base, GCH)], ii_v)
        pltpu.sync_copy(ij_h.at[pl.ds(base, GCH)], ij_v)

        @pl.loop(0, GCH // 16)
        def _grp(g):
            o = g * 16
            i16 = ii_v[pl.ds(o, 16)]
            j16 = ij_v[pl.ds(o, 16)]
            dx = plsc.load_gather(rx_v, [j16]) - plsc.load_gather(rx_v, [i16])
            dy = plsc.load_gather(ry_v, [j16]) - plsc.load_gather(ry_v, [i16])
            dz = plsc.load_gather(rz_v, [j16]) - plsc.load_gather(rz_v, [i16])
            ssq = dx * dx + dy * dy + dz * dz + 1e-12
            ib = plsc.bitcast(ssq, jnp.int32)
            y = plsc.bitcast(jnp.int32(0x5F3759DF) - (ib >> 1), jnp.float32)
            h = 0.5 * ssq
            y = y * (1.5 - h * y * y)
            y = y * (1.5 - h * y * y)
            y = y * (1.5 - h * y * y)
            dist = ssq * y
            u = dist * (1.0 / DD) + 0.5
            u = jnp.minimum(u, float(T - 1))
            lo_v[pl.ds(o, 16)] = u.astype(jnp.int32)

        pltpu.sync_copy(lo_v, lo_h.at[pl.ds(base, GCH)])


_SC_PARAMS = pltpu.CompilerParams(needs_layout_passes=False)

_geom = pl.kernel(
    _geom_body,
    out_type=jax.ShapeDtypeStruct((E,), jnp.int32),
    mesh=_mesh,
    compiler_params=_SC_PARAMS,
    scratch_types=[
        pltpu.VMEM((N,), jnp.float32),
        pltpu.VMEM((N,), jnp.float32),
        pltpu.VMEM((N,), jnp.float32),
        pltpu.VMEM((GCH,), jnp.int32),
        pltpu.VMEM((GCH,), jnp.int32),
        pltpu.VMEM((GCH,), jnp.int32),
    ],
)


def _msg_body(xf_h, wt_h, ii_h, ij_h, lo_h, zz_h, out_h,
              ij0, ij1, lo0, lo1, ii0, ii1, xj0, xj1, w0, w1,
              agg_sh, gsem0, gsem1, ssem0, ssem1):
    c = lax.axis_index("c")
    s = lax.axis_index("s")
    wid = s * NC + c
    ijv, lov, iiv = (ij0, ij1), (lo0, lo1), (ii0, ii1)
    xjv, wv = (xj0, xj1), (w0, w1)
    gsem, ssem = (gsem0, gsem1), (ssem0, ssem1)

    st = xj0.at[pl.ds(0, 64)]
    pltpu.sync_copy(zz_h, st)

    @pl.loop(0, RPT // 64)
    def _zero(k):
        pltpu.sync_copy(st, agg_sh.at[pl.ds(s * RPT + k * 64, 64)])
    plsc.subcore_barrier()

    base0 = wid * CPW * C

    def _load_idx(k, b):
        base = base0 + k * C
        pltpu.sync_copy(ij_h.at[pl.ds(base, C)], ijv[b])
        pltpu.sync_copy(lo_h.at[pl.ds(base, C)], lov[b])
        pltpu.sync_copy(ii_h.at[pl.ds(base, C)], iiv[b])

    def _gathers(b):
        d1 = pltpu.async_copy(xf_h.at[ijv[b]], xjv[b], gsem[b])
        d2 = pltpu.async_copy(wt_h.at[lov[b]], wv[b], gsem[b])
        return d1, d2

    def _mul(b):
        xb, wb = xjv[b], wv[b]

        @pl.loop(0, C)
        def _row(r):
            for kk in range(D // 16):
                sl = pl.ds(kk * 16, 16)
                xb[r, sl] = xb[r, sl] * wb[r, sl]

    def _scatter(b):
        return pltpu.async_copy(xjv[b], agg_sh.at[iiv[b]], ssem[b],
                                add=True)

    @pl.loop(0, CPW // 2)
    def _pair(t):
        _load_idx(2 * t, 0)
        _load_idx(2 * t + 1, 1)
        g0 = _gathers(0)
        g1 = _gathers(1)
        g0[0].wait()
        g0[1].wait()
        _mul(0)
        s0 = _scatter(0)
        g1[0].wait()
        g1[1].wait()
        _mul(1)
        s1 = _scatter(1)
        s0.wait()
        s1.wait()

    plsc.subcore_barrier()

    @pl.loop(0, RPT // 64)
    def _writeback(k):
        row = s * RPT + k * 64
        pltpu.sync_copy(agg_sh.at[pl.ds(row, 64)], st)
        pltpu.sync_copy(st, out_h.at[c, pl.ds(row, 64)])


_msg = pl.kernel(
    _msg_body,
    out_type=jax.ShapeDtypeStruct((NC, N_PAD, D), jnp.float32),
    mesh=_mesh,
    compiler_params=_SC_PARAMS,
    scratch_types=(
        [pltpu.VMEM((C,), jnp.int32)] * 6
        + [pltpu.VMEM((C, D), jnp.float32)] * 4
        + [pltpu.VMEM_SHARED((N_PAD, D), jnp.float32)]
        + [pltpu.SemaphoreType.DMA] * 4
    ),
)


def _tab_body(w1_ref, b1_ref, w2_ref, b2_ref, out_ref):
    t = pl.program_id(0)
    tb = out_ref.shape[1]
    d = (lax.broadcasted_iota(jnp.int32, (tb, 1), 0).astype(jnp.float32)
         + t * tb) * DD
    width = CUT / (NRBF - 1)
    offs = (lax.broadcasted_iota(jnp.int32, (1, NRBF), 1).astype(jnp.float32)
            * width)
    coeff = -0.5 / (width * width)
    fr = jnp.exp(coeff * (d - offs) ** 2)
    rc = 0.5 * (jnp.cos(d * (math.pi / CUT)) + 1.0)
    rc = rc * (d < CUT).astype(jnp.float32)
    for i in range(NI):
        h = _ssp(jnp.dot(fr, w1_ref[i], preferred_element_type=jnp.float32)
                 + b1_ref[i])
        w = jnp.dot(h, w2_ref[i], preferred_element_type=jnp.float32) + b2_ref[i]
        out_ref[i] = w * rc


TB = 2048


def _tables(fn_W1, fn_b1, fn_W2, fn_b2):
    return pl.pallas_call(
        _tab_body,
        out_shape=jax.ShapeDtypeStruct((NI, T, D), jnp.float32),
        grid=(T // TB,),
        in_specs=[
            pl.BlockSpec((NI, NRBF, NF), lambda t: (0, 0, 0)),
            pl.BlockSpec((NI, NF), lambda t: (0, 0)),
            pl.BlockSpec((NI, NF, NF), lambda t: (0, 0, 0)),
            pl.BlockSpec((NI, NF), lambda t: (0, 0)),
        ],
        out_specs=pl.BlockSpec((NI, TB, D), lambda t: (0, t, 0)),
    )(fn_W1, fn_b1, fn_W2, fn_b2)


def _emb_body(z_ref, emb_ref, w0_ref, x_ref, xf_ref):
    z = z_ref[0]
    onehot = (lax.broadcasted_iota(jnp.int32, (NB, ZMAX), 1)
              == z[:, None]).astype(jnp.float32)
    x = jnp.dot(onehot, emb_ref[...], preferred_element_type=jnp.float32)
    x_ref[...] = x
    xf_ref[...] = jnp.dot(x, w0_ref[...], preferred_element_type=jnp.float32)


def _embed(Z, emb, w0):
    return pl.pallas_call(
        _emb_body,
        out_shape=(jax.ShapeDtypeStruct((N, D), jnp.float32),
                   jax.ShapeDtypeStruct((N, NF), jnp.float32)),
        grid=(NGRID,),
        in_specs=[
            pl.BlockSpec((None, 1, NB), lambda n: (n, 0, 0)),
            pl.BlockSpec((ZMAX, D), lambda n: (0, 0)),
            pl.BlockSpec((D, NF), lambda n: (0, 0)),
        ],
        out_specs=(pl.BlockSpec((NB, D), lambda n: (n, 0)),
                   pl.BlockSpec((NB, NF), lambda n: (n, 0))),
    )(Z.reshape(NGRID, 1, NB), emb, w0)


def _upd_body(x_ref, ap_ref, w1_ref, b1_ref, w2_ref, b2_ref, wn_ref,
              xn_ref, xfn_ref):
    agg = ap_ref[0] + ap_ref[1]
    v = _ssp(jnp.dot(agg, w1_ref[...], preferred_element_type=jnp.float32)
             + b1_ref[...])
    v = jnp.dot(v, w2_ref[...], preferred_element_type=jnp.float32) + b2_ref[...]
    xn = x_ref[...] + v
    xn_ref[...] = xn
    if xfn_ref is not None:
        xfn_ref[...] = jnp.dot(xn, wn_ref[...],
                               preferred_element_type=jnp.float32)


def _update(x, aggp, w1, b1, w2, b2, wn):
    last = wn is None
    body = (functools.partial(_upd_body, xfn_ref=None) if last
            else _upd_body)
    out_shape = (jax.ShapeDtypeStruct((N, D), jnp.float32),)
    out_specs = (pl.BlockSpec((NB, D), lambda n: (n, 0)),)
    if not last:
        out_shape += (jax.ShapeDtypeStruct((N, NF), jnp.float32),)
        out_specs += (pl.BlockSpec((NB, NF), lambda n: (n, 0)),)
    res = pl.pallas_call(
        body,
        out_shape=out_shape,
        grid=(NGRID,),
        in_specs=[
            pl.BlockSpec((NB, D), lambda n: (n, 0)),
            pl.BlockSpec((NC, NB, D), lambda n: (0, n, 0)),
            pl.BlockSpec((NF, D), lambda n: (0, 0)),
            pl.BlockSpec((D,), lambda n: (0,)),
            pl.BlockSpec((D, D), lambda n: (0, 0)),
            pl.BlockSpec((D,), lambda n: (0,)),
            pl.BlockSpec((D, NF), lambda n: (0, 0)),
        ],
        out_specs=out_specs,
    )(x, aggp, w1, b1, w2, b2, wn if wn is not None else w2)
    return res if not last else (res[0], None)


def kernel(R, Z, idx_i, idx_j, emb, in2f_W, fn_W1, fn_b1, fn_W2, fn_b2,
           f2_W1, f2_b1, f2_W2, f2_b2):
    Rx = jnp.asarray(R[:, 0], jnp.float32)
    Ry = jnp.asarray(R[:, 1], jnp.float32)
    Rz = jnp.asarray(R[:, 2], jnp.float32)
    lo = _geom(Rx, Ry, Rz, idx_i, idx_j)
    wtab = _tables(fn_W1, fn_b1, fn_W2, fn_b2)
    x, xf = _embed(Z, emb, in2f_W[0])
    zz = jnp.zeros((64, D), jnp.float32)
    npad = NROW * C - E
    zpad = jnp.zeros((npad,), jnp.int32)
    ii1d = jnp.concatenate([idx_i, zpad])
    ij1d = jnp.concatenate([idx_j, zpad])
    lo1d = jnp.concatenate([lo, jnp.full((npad,), T - 1, jnp.int32)])
    for i in range(NI):
        aggp = _msg(xf, wtab[i], ii1d, ij1d, lo1d, zz)
        wn = in2f_W[i + 1] if i + 1 < NI else None
        x, xf = _update(x, aggp, f2_W1[i], f2_b1[i], f2_W2[i], f2_b2[i], wn)
    return x


# speedup vs baseline: 1.1763x; 1.1763x over previous
import functools
import math

import jax
import jax.numpy as jnp
import numpy as np
from jax import lax
from jax.experimental import pallas as pl
from jax.experimental.pallas import tpu as pltpu
from jax.experimental.pallas import tpu_sc as plsc

N = 10000
E = 320000
D = 128
NF = 128
NRBF = 50
NI = 3
CUT = 5.0
ZMAX = 400

T = 16384
DD = CUT / (T - 1)

NC = 2
NS = 16
NW = NC * NS
N_PAD = 10240
RPT = N_PAD // NS
C = 80
NROW = 4096
CPW = NROW // NW

EPW = E // NW
GCH = 2000
NG = EPW // GCH

NB = 2000
NGRID = N // NB

_mesh = plsc.VectorSubcoreMesh(core_axis_name="c", subcore_axis_name="s")


def _ssp(x):
    return jax.nn.softplus(x) - math.log(2.0)


def _geom_body(rx_h, ry_h, rz_h, ii_h, ij_h, lo_h,
               rx_v, ry_v, rz_v, ii_v, ij_v, lo_v):
    c = lax.axis_index("c")
    s = lax.axis_index("s")
    wid = s * NC + c
    pltpu.sync_copy(rx_h, rx_v)
    pltpu.sync_copy(ry_h, ry_v)
    pltpu.sync_copy(rz_h, rz_v)
    base0 = wid * EPW

    @pl.loop(0, NG)
    def _chunk(ch):
        base = base0 + ch * GCH
        pltpu.sync_copy(ii_h.at[pl.ds(base, GCH)], ii_v)
        pltpu.sync_copy(ij_h.at[pl.ds(base, GCH)], ij_v)

        @pl.loop(0, GCH // 16)
        def _grp(g):
            o = g * 16
            i16 = ii_v[pl.ds(o, 16)]
            j16 = ij_v[pl.ds(o, 16)]
            dx = plsc.load_gather(rx_v, [j16]) - plsc.load_gather(rx_v, [i16])
            dy = plsc.load_gather(ry_v, [j16]) - plsc.load_gather(ry_v, [i16])
            dz = plsc.load_gather(rz_v, [j16]) - plsc.load_gather(rz_v, [i16])
            ssq = dx * dx + dy * dy + dz * dz + 1e-12
            ib = plsc.bitcast(ssq, jnp.int32)
            y = plsc.bitcast(jnp.int32(0x5F3759DF) - (ib >> 1), jnp.float32)
            h = 0.5 * ssq
            y = y * (1.5 - h * y * y)
            y = y * (1.5 - h * y * y)
            y = y * (1.5 - h * y * y)
            dist = ssq * y
            u = dist * (1.0 / DD) + 0.5
            u = jnp.minimum(u, float(T - 1))
            lo_v[pl.ds(o, 16)] = u.astype(jnp.int32)

        pltpu.sync_copy(lo_v, lo_h.at[pl.ds(base, GCH)])


_SC_PARAMS = pltpu.CompilerParams(needs_layout_passes=False)

_geom = pl.kernel(
    _geom_body,
    out_type=jax.ShapeDtypeStruct((E,), jnp.int32),
    mesh=_mesh,
    compiler_params=_SC_PARAMS,
    scratch_types=[
        pltpu.VMEM((N,), jnp.float32),
        pltpu.VMEM((N,), jnp.float32),
        pltpu.VMEM((N,), jnp.float32),
        pltpu.VMEM((GCH,), jnp.int32),
        pltpu.VMEM((GCH,), jnp.int32),
        pltpu.VMEM((GCH,), jnp.int32),
    ],
)


def _msg_body(xf_h, wt_h, ii_h, ij_h, lo_h, zz_h, out_h,
              ij0, ij1, lo0, lo1, ii0, ii1, xj0, xj1, w0, w1,
              agg_sh, gsem0, gsem1, ssem0, ssem1):
    c = lax.axis_index("c")
    s = lax.axis_index("s")
    wid = s * NC + c
    ijv, lov, iiv = (ij0, ij1), (lo0, lo1), (ii0, ii1)
    xjv, wv = (xj0, xj1), (w0, w1)
    gsem, ssem = (gsem0, gsem1), (ssem0, ssem1)

    st = xj0.at[pl.ds(0, 64)]
    pltpu.sync_copy(zz_h, st)

    @pl.loop(0, RPT // 64)
    def _zero(k):
        pltpu.sync_copy(st, agg_sh.at[pl.ds(s * RPT + k * 64, 64)])
    plsc.subcore_barrier()

    base0 = wid * CPW * C

    def _load_idx(k, b):
        base = base0 + k * C
        pltpu.sync_copy(ij_h.at[pl.ds(base, C)], ijv[b])
        pltpu.sync_copy(lo_h.at[pl.ds(base, C)], lov[b])
        pltpu.sync_copy(ii_h.at[pl.ds(base, C)], iiv[b])

    def _gathers(b):
        d1 = pltpu.async_copy(xf_h.at[ijv[b]], xjv[b], gsem[b])
        d2 = pltpu.async_copy(wt_h.at[lov[b]], wv[b], gsem[b])
        return d1, d2

    def _mul(b):
        xb, wb = xjv[b], wv[b]

        @pl.loop(0, C)
        def _row(r):
            for kk in range(D // 16):
                sl = pl.ds(kk * 16, 16)
                xb[r, sl] = xb[r, sl] * wb[r, sl]

    def _scatter(b):
        return pltpu.async_copy(xjv[b], agg_sh.at[iiv[b]], ssem[b],
                                add=True)

    @pl.loop(0, CPW // 2)
    def _pair(t):
        _load_idx(2 * t, 0)
        _load_idx(2 * t + 1, 1)
        g0 = _gathers(0)
        g1 = _gathers(1)
        g0[0].wait()
        g0[1].wait()
        _mul(0)
        s0 = _scatter(0)
        g1[0].wait()
        g1[1].wait()
        _mul(1)
        s1 = _scatter(1)
        s0.wait()
        s1.wait()

    plsc.subcore_barrier()

    @pl.loop(0, RPT // 64)
    def _writeback(k):
        row = s * RPT + k * 64
        pltpu.sync_copy(agg_sh.at[pl.ds(row, 64)], st)
        pltpu.sync_copy(st, out_h.at[c, pl.ds(row, 64)])


_msg = pl.kernel(
    _msg_body,
    out_type=jax.ShapeDtypeStruct((NC, N_PAD, D), jnp.float32),
    mesh=_mesh,
    compiler_params=_SC_PARAMS,
    scratch_types=(
        [pltpu.VMEM((C,), jnp.int32)] * 6
        + [pltpu.VMEM((C, D), jnp.float32)] * 4
        + [pltpu.VMEM_SHARED((N_PAD, D), jnp.float32)]
        + [pltpu.SemaphoreType.DMA] * 4
    ),
)


def _tab_body(w1_ref, b1_ref, w2_ref, b2_ref, out_ref):
    t = pl.program_id(0)
    tb = out_ref.shape[1]
    d = (lax.broadcasted_iota(jnp.int32, (tb, 1), 0).astype(jnp.float32)
         + t * tb) * DD
    width = CUT / (NRBF - 1)
    offs = (lax.broadcasted_iota(jnp.int32, (1, NRBF), 1).astype(jnp.float32)
            * width)
    coeff = -0.5 / (width * width)
    fr = jnp.exp(coeff * (d - offs) ** 2)
    rc = 0.5 * (jnp.cos(d * (math.pi / CUT)) + 1.0)
    rc = rc * (d < CUT).astype(jnp.float32)
    for i in range(NI):
        h = _ssp(jnp.dot(fr, w1_ref[i], preferred_element_type=jnp.float32)
                 + b1_ref[i])
        w = jnp.dot(h, w2_ref[i], preferred_element_type=jnp.float32) + b2_ref[i]
        out_ref[i] = w * rc


TB = 2048


def _tables(fn_W1, fn_b1, fn_W2, fn_b2):
    return pl.pallas_call(
        _tab_body,
        out_shape=jax.ShapeDtypeStruct((NI, T, D), jnp.float32),
        grid=(T // TB,),
        in_specs=[
            pl.BlockSpec((NI, NRBF, NF), lambda t: (0, 0, 0)),
            pl.BlockSpec((NI, NF), lambda t: (0, 0)),
            pl.BlockSpec((NI, NF, NF), lambda t: (0, 0, 0)),
            pl.BlockSpec((NI, NF), lambda t: (0, 0)),
        ],
        out_specs=pl.BlockSpec((NI, TB, D), lambda t: (0, t, 0)),
    )(fn_W1, fn_b1, fn_W2, fn_b2)


def _emb_body(z_ref, emb_ref, w0_ref, x_ref, xf_ref):
    z = z_ref[0]
    onehot = (lax.broadcasted_iota(jnp.int32, (NB, ZMAX), 1)
              == z[:, None]).astype(jnp.float32)
    x = jnp.dot(onehot, emb_ref[...], preferred_element_type=jnp.float32)
    x_ref[...] = x
    xf_ref[...] = jnp.dot(x, w0_ref[...], preferred_element_type=jnp.float32)


def _embed(Z, emb, w0):
    return pl.pallas_call(
        _emb_body,
        out_shape=(jax.ShapeDtypeStruct((N, D), jnp.float32),
                   jax.ShapeDtypeStruct((N, NF), jnp.float32)),
        grid=(NGRID,),
        in_specs=[
            pl.BlockSpec((None, 1, NB), lambda n: (n, 0, 0)),
            pl.BlockSpec((ZMAX, D), lambda n: (0, 0)),
            pl.BlockSpec((D, NF), lambda n: (0, 0)),
        ],
        out_specs=(pl.BlockSpec((NB, D), lambda n: (n, 0)),
                   pl.BlockSpec((NB, NF), lambda n: (n, 0))),
    )(Z.reshape(NGRID, 1, NB), emb, w0)


def _upd_body(x_ref, ap_ref, w1_ref, b1_ref, w2_ref, b2_ref, wn_ref,
              xn_ref, xfn_ref):
    agg = ap_ref[0] + ap_ref[1]
    v = _ssp(jnp.dot(agg, w1_ref[...], preferred_element_type=jnp.float32)
             + b1_ref[...])
    v = jnp.dot(v, w2_ref[...], preferred_element_type=jnp.float32) + b2_ref[...]
    xn = x_ref[...] + v
    xn_ref[...] = xn
    if xfn_ref is not None:
        xfn_ref[...] = jnp.dot(xn, wn_ref[...],
                               preferred_element_type=jnp.float32)


def _update(x, aggp, w1, b1, w2, b2, wn):
    last = wn is None
    body = (functools.partial(_upd_body, xfn_ref=None) if last
            else _upd_body)
    out_shape = (jax.ShapeDtypeStruct((N, D), jnp.float32),)
    out_specs = (pl.BlockSpec((NB, D), lambda n: (n, 0)),)
    if not last:
        out_shape += (jax.ShapeDtypeStruct((N, NF), jnp.float32),)
        out_specs += (pl.BlockSpec((NB, NF), lambda n: (n, 0)),)
    res = pl.pallas_call(
        body,
        out_shape=out_shape,
        grid=(NGRID,),
        in_specs=[
            pl.BlockSpec((NB, D), lambda n: (n, 0)),
            pl.BlockSpec((NC, NB, D), lambda n: (0, n, 0)),
            pl.BlockSpec((NF, D), lambda n: (0, 0)),
            pl.BlockSpec((D,), lambda n: (0,)),
            pl.BlockSpec((D, D), lambda n: (0, 0)),
            pl.BlockSpec((D,), lambda n: (0,)),
            pl.BlockSpec((D, NF), lambda n: (0, 0)),
        ],
        out_specs=out_specs,
    )(x, aggp, w1, b1, w2, b2, wn if wn is not None else w2)
    return res if not last else (res[0], None)


def kernel(R, Z, idx_i, idx_j, emb, in2f_W, fn_W1, fn_b1, fn_W2, fn_b2,
           f2_W1, f2_b1, f2_W2, f2_b2):
    Rx = jnp.asarray(R[:, 0], jnp.float32)
    Ry = jnp.asarray(R[:, 1], jnp.float32)
    Rz = jnp.asarray(R[:, 2], jnp.float32)
    lo = _geom(Rx, Ry, Rz, idx_i, idx_j)
    wtab = _tables(fn_W1, fn_b1, fn_W2, fn_b2)
    x, xf = _embed(Z, emb, in2f_W[0])
    zz = jnp.zeros((64, D), jnp.float32)
    npad = NROW * C - E
    pad_ii = N + (jnp.arange(npad, dtype=jnp.int32) % (N_PAD - N))
    pad_ij = jnp.arange(npad, dtype=jnp.int32) % N
    ii1d = jnp.concatenate([idx_i, pad_ii])
    ij1d = jnp.concatenate([idx_j, pad_ij])
    lo1d = jnp.concatenate([lo, jnp.full((npad,), T - 1, jnp.int32)])
    for i in range(NI):
        aggp = _msg(xf, wtab[i], ii1d, ij1d, lo1d, zz)
        wn = in2f_W[i + 1] if i + 1 < NI else None
        x, xf = _update(x, aggp, f2_W1[i], f2_b1[i], f2_W2[i], f2_b2[i], wn)
    return x


# speedup vs baseline: 2.5433x; 2.1622x over previous
import functools
import math

import jax
import jax.numpy as jnp
import numpy as np
from jax import lax
from jax.experimental import pallas as pl
from jax.experimental.pallas import tpu as pltpu
from jax.experimental.pallas import tpu_sc as plsc

N = 10000
E = 320000
D = 128
NF = 128
NRBF = 50
NI = 3
CUT = 5.0
ZMAX = 400

T = 16384
DD = CUT / (T - 1)

NC = 2
NS = 16
NW = NC * NS
N_PAD = 10240
RPT = N_PAD // NS
C = 80
CPW = E // (NW * C)

EPW = E // NW
GCH = 2000
NG = EPW // GCH

NB = 2000
NGRID = N // NB

_mesh = plsc.VectorSubcoreMesh(core_axis_name="c", subcore_axis_name="s")


def _ssp(x):
    return jax.nn.softplus(x) - math.log(2.0)


def _geom_body(rx_h, ry_h, rz_h, ii_h, ij_h, lo_h,
               rx_v, ry_v, rz_v, ii_v, ij_v, lo_v):
    c = lax.axis_index("c")
    s = lax.axis_index("s")
    wid = s * NC + c
    pltpu.sync_copy(rx_h, rx_v)
    pltpu.sync_copy(ry_h, ry_v)
    pltpu.sync_copy(rz_h, rz_v)
    base0 = wid * EPW

    @pl.loop(0, NG)
    def _chunk(ch):
        base = base0 + ch * GCH
        pltpu.sync_copy(ii_h.at[pl.ds(base, GCH)], ii_v)
        pltpu.sync_copy(ij_h.at[pl.ds(base, GCH)], ij_v)

        @pl.loop(0, GCH // 16)
        def _grp(g):
            o = g * 16
            i16 = ii_v[pl.ds(o, 16)]
            j16 = ij_v[pl.ds(o, 16)]
            dx = plsc.load_gather(rx_v, [j16]) - plsc.load_gather(rx_v, [i16])
            dy = plsc.load_gather(ry_v, [j16]) - plsc.load_gather(ry_v, [i16])
            dz = plsc.load_gather(rz_v, [j16]) - plsc.load_gather(rz_v, [i16])
            ssq = dx * dx + dy * dy + dz * dz + 1e-12
            ib = plsc.bitcast(ssq, jnp.int32)
            y = plsc.bitcast(jnp.int32(0x5F3759DF) - (ib >> 1), jnp.float32)
            h = 0.5 * ssq
            y = y * (1.5 - h * y * y)
            y = y * (1.5 - h * y * y)
            y = y * (1.5 - h * y * y)
            dist = ssq * y
            u = dist * (1.0 / DD) + 0.5
            u = jnp.minimum(u, float(T - 1))
            lo_v[pl.ds(o, 16)] = u.astype(jnp.int32)

        pltpu.sync_copy(lo_v, lo_h.at[pl.ds(base, GCH)])


_SC_PARAMS = pltpu.CompilerParams(needs_layout_passes=False)

_geom = pl.kernel(
    _geom_body,
    out_type=jax.ShapeDtypeStruct((E,), jnp.int32),
    mesh=_mesh,
    compiler_params=_SC_PARAMS,
    scratch_types=[
        pltpu.VMEM((N,), jnp.float32),
        pltpu.VMEM((N,), jnp.float32),
        pltpu.VMEM((N,), jnp.float32),
        pltpu.VMEM((GCH,), jnp.int32),
        pltpu.VMEM((GCH,), jnp.int32),
        pltpu.VMEM((GCH,), jnp.int32),
    ],
)


def _msg_body(xf_h, wt_h, ii_h, ij_h, lo_h, zz_h, out_h,
              ij0, ij1, lo0, lo1, ii0, ii1, xj0, xj1, w0, w1,
              agg_sh, gsem0, gsem1, ssem0, ssem1):
    c = lax.axis_index("c")
    s = lax.axis_index("s")
    wid = s * NC + c
    ijv, lov, iiv = (ij0, ij1), (lo0, lo1), (ii0, ii1)
    xjv, wv = (xj0, xj1), (w0, w1)
    gsem, ssem = (gsem0, gsem1), (ssem0, ssem1)

    st = xj0.at[pl.ds(0, 64)]
    pltpu.sync_copy(zz_h, st)

    @pl.loop(0, RPT // 64)
    def _zero(k):
        pltpu.sync_copy(st, agg_sh.at[pl.ds(s * RPT + k * 64, 64)])
    plsc.subcore_barrier()

    base0 = wid * CPW * C

    def _issue(k, b):
        base = base0 + k * C
        pltpu.sync_copy(ij_h.at[pl.ds(base, C)], ijv[b])
        pltpu.sync_copy(lo_h.at[pl.ds(base, C)], lov[b])
        pltpu.sync_copy(ii_h.at[pl.ds(base, C)], iiv[b])
        d1 = pltpu.async_copy(xf_h.at[ijv[b]], xjv[b], gsem[b])
        d2 = pltpu.async_copy(wt_h.at[lov[b]], wv[b], gsem[b])
        return d1, d2

    def _mul(b):
        xb, wb = xjv[b], wv[b]

        @pl.loop(0, C)
        def _row(r):
            for kk in range(D // 16):
                sl = pl.ds(kk * 16, 16)
                xb[r, sl] = xb[r, sl] * wb[r, sl]

    def _scatter(b):
        return pltpu.async_copy(xjv[b], agg_sh.at[iiv[b]], ssem[b],
                                add=True)

    @pl.loop(0, CPW // 2)
    def _pair(t):
        g0 = _issue(2 * t, 0)
        g1 = _issue(2 * t + 1, 1)
        g0[0].wait()
        g0[1].wait()
        _mul(0)
        s0 = _scatter(0)
        g1[0].wait()
        g1[1].wait()
        _mul(1)
        s1 = _scatter(1)
        s0.wait()
        s1.wait()

    g0 = _issue(CPW - 1, 0)
    g0[0].wait()
    g0[1].wait()
    _mul(0)
    s0 = _scatter(0)
    s0.wait()
    plsc.subcore_barrier()

    @pl.loop(0, RPT // 64)
    def _writeback(k):
        row = s * RPT + k * 64
        pltpu.sync_copy(agg_sh.at[pl.ds(row, 64)], st)
        pltpu.sync_copy(st, out_h.at[c, pl.ds(row, 64)])


_msg = pl.kernel(
    _msg_body,
    out_type=jax.ShapeDtypeStruct((NC, N_PAD, D), jnp.float32),
    mesh=_mesh,
    compiler_params=_SC_PARAMS,
    scratch_types=(
        [pltpu.VMEM((C,), jnp.int32)] * 6
        + [pltpu.VMEM((C, D), jnp.float32)] * 4
        + [pltpu.VMEM_SHARED((N_PAD, D), jnp.float32)]
        + [pltpu.SemaphoreType.DMA] * 4
    ),
)


def _tab_body(w1_ref, b1_ref, w2_ref, b2_ref, out_ref):
    t = pl.program_id(0)
    tb = out_ref.shape[1]
    d = (lax.broadcasted_iota(jnp.int32, (tb, 1), 0).astype(jnp.float32)
         + t * tb) * DD
    width = CUT / (NRBF - 1)
    offs = (lax.broadcasted_iota(jnp.int32, (1, NRBF), 1).astype(jnp.float32)
            * width)
    coeff = -0.5 / (width * width)
    fr = jnp.exp(coeff * (d - offs) ** 2)
    rc = 0.5 * (jnp.cos(d * (math.pi / CUT)) + 1.0)
    rc = rc * (d < CUT).astype(jnp.float32)
    for i in range(NI):
        h = _ssp(jnp.dot(fr, w1_ref[i], preferred_element_type=jnp.float32)
                 + b1_ref[i])
        w = jnp.dot(h, w2_ref[i], preferred_element_type=jnp.float32) + b2_ref[i]
        out_ref[i] = w * rc


TB = 2048


def _tables(fn_W1, fn_b1, fn_W2, fn_b2):
    return pl.pallas_call(
        _tab_body,
        out_shape=jax.ShapeDtypeStruct((NI, T, D), jnp.float32),
        grid=(T // TB,),
        in_specs=[
            pl.BlockSpec((NI, NRBF, NF), lambda t: (0, 0, 0)),
            pl.BlockSpec((NI, NF), lambda t: (0, 0)),
            pl.BlockSpec((NI, NF, NF), lambda t: (0, 0, 0)),
            pl.BlockSpec((NI, NF), lambda t: (0, 0)),
        ],
        out_specs=pl.BlockSpec((NI, TB, D), lambda t: (0, t, 0)),
    )(fn_W1, fn_b1, fn_W2, fn_b2)


def _emb_body(z_ref, emb_ref, w0_ref, x_ref, xf_ref):
    z = z_ref[0]
    onehot = (lax.broadcasted_iota(jnp.int32, (NB, ZMAX), 1)
              == z[:, None]).astype(jnp.float32)
    x = jnp.dot(onehot, emb_ref[...], preferred_element_type=jnp.float32)
    x_ref[...] = x
    xf_ref[...] = jnp.dot(x, w0_ref[...], preferred_element_type=jnp.float32)


def _embed(Z, emb, w0):
    return pl.pallas_call(
        _emb_body,
        out_shape=(jax.ShapeDtypeStruct((N, D), jnp.float32),
                   jax.ShapeDtypeStruct((N, NF), jnp.float32)),
        grid=(NGRID,),
        in_specs=[
            pl.BlockSpec((None, 1, NB), lambda n: (n, 0, 0)),
            pl.BlockSpec((ZMAX, D), lambda n: (0, 0)),
            pl.BlockSpec((D, NF), lambda n: (0, 0)),
        ],
        out_specs=(pl.BlockSpec((NB, D), lambda n: (n, 0)),
                   pl.BlockSpec((NB, NF), lambda n: (n, 0))),
    )(Z.reshape(NGRID, 1, NB), emb, w0)


def _upd_body(x_ref, ap_ref, w1_ref, b1_ref, w2_ref, b2_ref, wn_ref,
              xn_ref, xfn_ref):
    agg = ap_ref[0] + ap_ref[1]
    v = _ssp(jnp.dot(agg, w1_ref[...], preferred_element_type=jnp.float32)
             + b1_ref[...])
    v = jnp.dot(v, w2_ref[...], preferred_element_type=jnp.float32) + b2_ref[...]
    xn = x_ref[...] + v
    xn_ref[...] = xn
    if xfn_ref is not None:
        xfn_ref[...] = jnp.dot(xn, wn_ref[...],
                               preferred_element_type=jnp.float32)


def _update(x, aggp, w1, b1, w2, b2, wn):
    last = wn is None
    body = (functools.partial(_upd_body, xfn_ref=None) if last
            else _upd_body)
    out_shape = (jax.ShapeDtypeStruct((N, D), jnp.float32),)
    out_specs = (pl.BlockSpec((NB, D), lambda n: (n, 0)),)
    if not last:
        out_shape += (jax.ShapeDtypeStruct((N, NF), jnp.float32),)
        out_specs += (pl.BlockSpec((NB, NF), lambda n: (n, 0)),)
    res = pl.pallas_call(
        body,
        out_shape=out_shape,
        grid=(NGRID,),
        in_specs=[
            pl.BlockSpec((NB, D), lambda n: (n, 0)),
            pl.BlockSpec((NC, NB, D), lambda n: (0, n, 0)),
            pl.BlockSpec((NF, D), lambda n: (0, 0)),
            pl.BlockSpec((D,), lambda n: (0,)),
            pl.BlockSpec((D, D), lambda n: (0, 0)),
            pl.BlockSpec((D,), lambda n: (0,)),
            pl.BlockSpec((D, NF), lambda n: (0, 0)),
        ],
        out_specs=out_specs,
    )(x, aggp, w1, b1, w2, b2, wn if wn is not None else w2)
    return res if not last else (res[0], None)


def kernel(R, Z, idx_i, idx_j, emb, in2f_W, fn_W1, fn_b1, fn_W2, fn_b2,
           f2_W1, f2_b1, f2_W2, f2_b2):
    Rx = jnp.asarray(R[:, 0], jnp.float32)
    Ry = jnp.asarray(R[:, 1], jnp.float32)
    Rz = jnp.asarray(R[:, 2], jnp.float32)
    lo = _geom(Rx, Ry, Rz, idx_i, idx_j)
    wtab = _tables(fn_W1, fn_b1, fn_W2, fn_b2)
    x, xf = _embed(Z, emb, in2f_W[0])
    zz = jnp.zeros((64, D), jnp.float32)
    for i in range(NI):
        aggp = _msg(xf, wtab[i], idx_i, idx_j, lo, zz)
        wn = in2f_W[i + 1] if i + 1 < NI else None
        x, xf = _update(x, aggp, f2_W1[i], f2_b1[i], f2_W2[i], f2_b2[i], wn)
    return x
